# baseline (speedup 1.0000x reference)
import functools

import jax
import jax.numpy as jnp
from jax import lax
from jax.experimental import pallas as pl
from jax.experimental.pallas import tpu as pltpu
from jax.experimental.pallas import tpu_sc as plsc

N = 10000
E = 320000
D = 128
L = 3

NC = 2
NS = 16
NW = NC * NS

B = 5000
KQ = 128
SLOTS = 48
NQ = 4
CAPQ = NS * SLOTS * KQ

HS_ROWS = 5120
HSTRIPE = HS_ROWS // NS
AGG_ROWS = 5248
ASTRIPE = AGG_ROWS // NS
DUMMY = 5100

BLK = 1000
NB = N // BLK


def _agg_body(h_hbm, src_hbm, dst_hbm, zeros_hbm, out_hbm,
              src_t, dst_t, rows0, rows1, h_sh, agg_sh,
              hsem, isem, zsem, gsem0, gsem1, ssem0, ssem1):
    c = lax.axis_index("c")
    s = lax.axis_index("s")
    src_off = jnp.minimum(c * B + s * HSTRIPE, N - HSTRIPE)
    dst_off = src_off - c * B
    hcp = pltpu.make_async_copy(h_hbm.at[pl.ds(src_off, HSTRIPE)],
                                h_sh.at[pl.ds(dst_off, HSTRIPE)], hsem)
    hcp.start()

    bufs = (rows0, rows1)
    gsems = (gsem0, gsem1)
    ssems = (ssem0, ssem1)

    def start_gather(b, j):
        pltpu.async_copy(h_sh.at[src_t.at[j]], bufs[b], gsems[b])

    def wait_gather(b, j):
        pltpu.make_async_copy(h_sh.at[src_t.at[j]], bufs[b], gsems[b]).wait()

    def start_scatter(b, j):
        pltpu.async_copy(bufs[b], agg_sh.at[dst_t.at[j]], ssems[b], add=True)

    def wait_scatter(b, j):
        pltpu.make_async_copy(bufs[b], agg_sh.at[dst_t.at[j]],
                              ssems[b]).wait()

    def run_pass(p):
        q = c * 2 + p
        az = agg_sh.at[pl.ds(s * ASTRIPE, ASTRIPE)]
        zc = pltpu.make_async_copy(zeros_hbm, az, zsem)
        sc_ = pltpu.make_async_copy(src_hbm.at[q].at[s], src_t, isem)
        dc_ = pltpu.make_async_copy(dst_hbm.at[q].at[s], dst_t, isem)
        zc.start()
        sc_.start()
        dc_.start()
        zc.wait()
        sc_.wait()
        dc_.wait()
        plsc.subcore_barrier()

        start_gather(0, 0)

        def body(j, carry):
            def step(cur, oth):
                wait_gather(cur, j)
                start_scatter(cur, j)

                @pl.when(j >= 1)
                def _():
                    wait_scatter(oth, j - 1)

                @pl.when(j + 1 < SLOTS)
                def _():
                    start_gather(oth, j + 1)

            @pl.when(j % 2 == 0)
            def _():
                step(0, 1)

            @pl.when(j % 2 == 1)
            def _():
                step(1, 0)

            return carry

        lax.fori_loop(0, SLOTS, body, 0)
        wait_scatter((SLOTS - 1) % 2, SLOTS - 1)
        plsc.subcore_barrier()
        off = q * AGG_ROWS + s * ASTRIPE
        pltpu.sync_copy(agg_sh.at[pl.ds(s * ASTRIPE, ASTRIPE)],
                        out_hbm.at[pl.ds(off, ASTRIPE)])

    hcp.wait()
    run_pass(0)
    run_pass(1)


@functools.cache
def _agg_kernel():
    return pl.kernel(
        _agg_body,
        out_type=jax.ShapeDtypeStruct((NC * 2 * AGG_ROWS, D), jnp.float32),
        mesh=plsc.VectorSubcoreMesh(core_axis_name="c", subcore_axis_name="s",
                                    num_cores=NC, num_subcores=NS),
        scratch_types=[
            pltpu.VMEM((SLOTS, KQ), jnp.int32),
            pltpu.VMEM((SLOTS, KQ), jnp.int32),
            pltpu.VMEM((KQ, D), jnp.float32),
            pltpu.VMEM((KQ, D), jnp.float32),
            pltpu.VMEM_SHARED((HS_ROWS, D), jnp.float32),
            pltpu.VMEM_SHARED((AGG_ROWS, D), jnp.float32),
            pltpu.SemaphoreType.DMA,
            pltpu.SemaphoreType.DMA,
            pltpu.SemaphoreType.DMA,
            pltpu.SemaphoreType.DMA,
            pltpu.SemaphoreType.DMA,
            pltpu.SemaphoreType.DMA,
            pltpu.SemaphoreType.DMA,
        ],
    )


def _agg(h, tbl_src, tbl_dst, zeros_a):
    return _agg_kernel()(h, tbl_src, tbl_dst, zeros_a)


def _lin_body(x_ref, w_ref, b_ref, o_ref):
    o_ref[...] = (
        jnp.dot(x_ref[...], w_ref[...], preferred_element_type=jnp.float32)
        + b_ref[...]
    )


def _linear(x, w, b):
    return pl.pallas_call(
        _lin_body,
        grid=(NB,),
        in_specs=[
            pl.BlockSpec((BLK, D), lambda i: (i, 0)),
            pl.BlockSpec((D, D), lambda i: (0, 0)),
            pl.BlockSpec((1, D), lambda i: (0, 0)),
        ],
        out_specs=pl.BlockSpec((BLK, D), lambda i: (i, 0)),
        out_shape=jax.ShapeDtypeStruct((N, D), jnp.float32),
    )(x, w, b)


def _mlp_body(h_ref, agg_ref, w1_ref, b1_ref, w2_ref, b2_ref,
              z_ref, stats_ref, acc_ref):
    i = pl.program_id(0)
    z = h_ref[...] + agg_ref[0, 0] + agg_ref[1, 0]
    z = jnp.maximum(
        jnp.dot(z, w1_ref[...], preferred_element_type=jnp.float32)
        + b1_ref[...], 0.0)
    z = jnp.maximum(
        jnp.dot(z, w2_ref[...], preferred_element_type=jnp.float32)
        + b2_ref[...], 0.0)
    z_ref[...] = z

    @pl.when(i == 0)
    def _():
        acc_ref[...] = jnp.zeros_like(acc_ref)

    acc_ref[0:1] += jnp.sum(z, axis=0, keepdims=True)
    acc_ref[1:2] += jnp.sum(z * z, axis=0, keepdims=True)
    stats_ref[...] = acc_ref[...]


def _mlp(h, parts, w1, b1, w2, b2):
    nblk = B // BLK
    return pl.pallas_call(
        _mlp_body,
        grid=(NB,),
        in_specs=[
            pl.BlockSpec((BLK, D), lambda i: (i, 0)),
            pl.BlockSpec((NC, 1, BLK, D),
                         lambda i: (0, i // nblk, i % nblk, 0)),
            pl.BlockSpec((D, D), lambda i: (0, 0)),
            pl.BlockSpec((1, D), lambda i: (0, 0)),
            pl.BlockSpec((D, D), lambda i: (0, 0)),
            pl.BlockSpec((1, D), lambda i: (0, 0)),
        ],
        out_specs=[
            pl.BlockSpec((BLK, D), lambda i: (i, 0)),
            pl.BlockSpec((2, D), lambda i: (0, 0)),
        ],
        out_shape=[
            jax.ShapeDtypeStruct((N, D), jnp.float32),
            jax.ShapeDtypeStruct((2, D), jnp.float32),
        ],
        scratch_shapes=[pltpu.VMEM((2, D), jnp.float32)],
    )(h, parts, w1, b1, w2, b2)


def _bn_body(z_ref, stats_ref, g_ref, be_ref, o_ref):
    inv_n = jnp.float32(1.0 / N)
    mean = stats_ref[0:1] * inv_n
    var = stats_ref[1:2] * inv_n - mean * mean
    scale = g_ref[...] * lax.rsqrt(var + 1e-5)
    o_ref[...] = jnp.tanh((z_ref[...] - mean) * scale + be_ref[...])


def _bn(z, stats, g, be):
    return pl.pallas_call(
        _bn_body,
        grid=(NB,),
        in_specs=[
            pl.BlockSpec((BLK, D), lambda i: (i, 0)),
            pl.BlockSpec((2, D), lambda i: (0, 0)),
            pl.BlockSpec((1, D), lambda i: (0, 0)),
            pl.BlockSpec((1, D), lambda i: (0, 0)),
        ],
        out_specs=pl.BlockSpec((BLK, D), lambda i: (i, 0)),
        out_shape=jax.ShapeDtypeStruct((N, D), jnp.float32),
    )(z, stats, g, be)


def _build_tables(edge_index):
    src = edge_index[0]
    dst = edge_index[1]
    qid = (src >= B).astype(jnp.int32) * 2 + (dst >= B).astype(jnp.int32)
    lsrc = src - (qid // 2) * B
    ldst = dst - (qid % 2) * B
    onehot = (qid[:, None] == jnp.arange(NQ, dtype=jnp.int32)[None, :])
    within = jnp.cumsum(onehot.astype(jnp.int32), axis=0)
    pos = jnp.take_along_axis(within, qid[:, None], axis=1)[:, 0] - 1
    ch = pos // KQ
    lane = pos % KQ
    tile = ch % NS
    slot = ch // NS
    tix = ((qid * NS + tile) * SLOTS + slot) * KQ + lane
    tbl_src = jnp.zeros((NQ * NS * SLOTS * KQ,), jnp.int32).at[tix].set(lsrc)
    tbl_dst = jnp.full((NQ * NS * SLOTS * KQ,), DUMMY,
                       jnp.int32).at[tix].set(ldst)
    return (tbl_src.reshape(NQ, NS, SLOTS, KQ),
            tbl_dst.reshape(NQ, NS, SLOTS, KQ))


def kernel(x, edge_index, W0, b0, W1, B1, W2, B2, G, Be):
    tbl_src, tbl_dst = _build_tables(edge_index)
    zeros_a = jnp.zeros((ASTRIPE, D), jnp.float32)

    h = _linear(x, W0, b0.reshape(1, D))
    outs = [x]
    for l in range(L):
        parts = _agg(h, tbl_src, tbl_dst,
                     zeros_a).reshape(NC, 2, AGG_ROWS, D)
        z, stats = _mlp(h, parts, W1[l], B1[l].reshape(1, D),
                        W2[l], B2[l].reshape(1, D))
        h = _bn(z, stats, G[l].reshape(1, D), Be[l].reshape(1, D))
        outs.append(h)
    return tuple(outs)


# speedup vs baseline: 3.0520x; 3.0520x over previous
import functools

import jax
import jax.numpy as jnp
from jax import lax
from jax.experimental import pallas as pl
from jax.experimental.pallas import tpu as pltpu
from jax.experimental.pallas import tpu_sc as plsc

N = 10000
E = 320000
D = 128
L = 3

NC = 2
NS = 16
NW = NC * NS

B = 5000
KQ = 128
SLOTS = 48
NQ = 4
CAPQ = NS * SLOTS * KQ

HS_ROWS = 5120
HSTRIPE = HS_ROWS // NS
AGG_ROWS = 5248
ASTRIPE = AGG_ROWS // NS
DUMMY = 5100

BLK = 1000
NB = N // BLK


def _agg_body(h_hbm, src_hbm, dst_hbm, zeros_hbm, out_hbm,
              src_t, dst_t, rows0, rows1, h_sh, agg_sh,
              hsem, isem, zsem, gsem0, gsem1, ssem0, ssem1):
    c = lax.axis_index("c")
    s = lax.axis_index("s")
    hoff = c * B + s * 312

    def _h_copy(rows):
        return pltpu.make_async_copy(h_hbm.at[pl.ds(hoff, rows)],
                                     h_sh.at[pl.ds(s * 312, rows)], hsem)

    @pl.when(s < NS - 1)
    def _():
        _h_copy(312).start()

    @pl.when(s == NS - 1)
    def _():
        _h_copy(320).start()

    bufs = (rows0, rows1)
    gsems = (gsem0, gsem1)
    ssems = (ssem0, ssem1)

    def start_gather(b, j):
        pltpu.async_copy(h_sh.at[src_t.at[j]], bufs[b], gsems[b])

    def wait_gather(b, j):
        pltpu.make_async_copy(h_sh.at[src_t.at[j]], bufs[b], gsems[b]).wait()

    def start_scatter(b, j):
        pltpu.async_copy(bufs[b], agg_sh.at[dst_t.at[j]], ssems[b], add=True)

    def wait_scatter(b, j):
        pltpu.make_async_copy(bufs[b], agg_sh.at[dst_t.at[j]],
                              ssems[b]).wait()

    def run_pass(p):
        q = c * 2 + p
        az = agg_sh.at[pl.ds(s * ASTRIPE, ASTRIPE)]
        zc = pltpu.make_async_copy(zeros_hbm, az, zsem)
        sc_ = pltpu.make_async_copy(src_hbm.at[q].at[s], src_t, isem)
        dc_ = pltpu.make_async_copy(dst_hbm.at[q].at[s], dst_t, isem)
        zc.start()
        sc_.start()
        dc_.start()
        zc.wait()
        sc_.wait()
        dc_.wait()
        plsc.subcore_barrier()

        start_gather(0, 0)

        def body(j, carry):
            def step(cur, oth):
                wait_gather(cur, j)
                start_scatter(cur, j)

                @pl.when(j >= 1)
                def _():
                    wait_scatter(oth, j - 1)

                @pl.when(j + 1 < SLOTS)
                def _():
                    start_gather(oth, j + 1)

            @pl.when(j % 2 == 0)
            def _():
                step(0, 1)

            @pl.when(j % 2 == 1)
            def _():
                step(1, 0)

            return carry

        lax.fori_loop(0, SLOTS, body, 0)
        wait_scatter((SLOTS - 1) % 2, SLOTS - 1)
        plsc.subcore_barrier()
        off = q * AGG_ROWS + s * ASTRIPE
        pltpu.sync_copy(agg_sh.at[pl.ds(s * ASTRIPE, ASTRIPE)],
                        out_hbm.at[pl.ds(off, ASTRIPE)])

    @pl.when(s < NS - 1)
    def _():
        _h_copy(312).wait()

    @pl.when(s == NS - 1)
    def _():
        _h_copy(320).wait()

    run_pass(0)
    run_pass(1)


@functools.cache
def _agg_kernel():
    return pl.kernel(
        _agg_body,
        out_type=jax.ShapeDtypeStruct((NC * 2 * AGG_ROWS, D), jnp.float32),
        mesh=plsc.VectorSubcoreMesh(core_axis_name="c", subcore_axis_name="s",
                                    num_cores=NC, num_subcores=NS),
        scratch_types=[
            pltpu.VMEM((SLOTS, KQ), jnp.int32),
            pltpu.VMEM((SLOTS, KQ), jnp.int32),
            pltpu.VMEM((KQ, D), jnp.float32),
            pltpu.VMEM((KQ, D), jnp.float32),
            pltpu.VMEM_SHARED((HS_ROWS, D), jnp.float32),
            pltpu.VMEM_SHARED((AGG_ROWS, D), jnp.float32),
            pltpu.SemaphoreType.DMA,
            pltpu.SemaphoreType.DMA,
            pltpu.SemaphoreType.DMA,
            pltpu.SemaphoreType.DMA,
            pltpu.SemaphoreType.DMA,
            pltpu.SemaphoreType.DMA,
            pltpu.SemaphoreType.DMA,
        ],
    )


def _agg(h, tbl_src, tbl_dst, zeros_a):
    return _agg_kernel()(h, tbl_src, tbl_dst, zeros_a)


def _lin_body(x_ref, w_ref, b_ref, o_ref):
    o_ref[...] = (
        jnp.dot(x_ref[...], w_ref[...], preferred_element_type=jnp.float32)
        + b_ref[...]
    )


def _linear(x, w, b):
    return pl.pallas_call(
        _lin_body,
        grid=(NB,),
        in_specs=[
            pl.BlockSpec((BLK, D), lambda i: (i, 0)),
            pl.BlockSpec((D, D), lambda i: (0, 0)),
            pl.BlockSpec((1, D), lambda i: (0, 0)),
        ],
        out_specs=pl.BlockSpec((BLK, D), lambda i: (i, 0)),
        out_shape=jax.ShapeDtypeStruct((N, D), jnp.float32),
    )(x, w, b)


def _mlp_body(h_ref, agg_ref, w1_ref, b1_ref, w2_ref, b2_ref,
              z_ref, stats_ref, acc_ref):
    i = pl.program_id(0)
    z = h_ref[...] + agg_ref[0, 0] + agg_ref[1, 0]
    z = jnp.maximum(
        jnp.dot(z, w1_ref[...], preferred_element_type=jnp.float32)
        + b1_ref[...], 0.0)
    z = jnp.maximum(
        jnp.dot(z, w2_ref[...], preferred_element_type=jnp.float32)
        + b2_ref[...], 0.0)
    z_ref[...] = z

    @pl.when(i == 0)
    def _():
        acc_ref[...] = jnp.zeros_like(acc_ref)

    acc_ref[0:1] += jnp.sum(z, axis=0, keepdims=True)
    acc_ref[1:2] += jnp.sum(z * z, axis=0, keepdims=True)
    stats_ref[...] = acc_ref[...]


def _mlp(h, parts, w1, b1, w2, b2):
    nblk = B // BLK
    return pl.pallas_call(
        _mlp_body,
        grid=(NB,),
        in_specs=[
            pl.BlockSpec((BLK, D), lambda i: (i, 0)),
            pl.BlockSpec((NC, 1, BLK, D),
                         lambda i: (0, i // nblk, i % nblk, 0)),
            pl.BlockSpec((D, D), lambda i: (0, 0)),
            pl.BlockSpec((1, D), lambda i: (0, 0)),
            pl.BlockSpec((D, D), lambda i: (0, 0)),
            pl.BlockSpec((1, D), lambda i: (0, 0)),
        ],
        out_specs=[
            pl.BlockSpec((BLK, D), lambda i: (i, 0)),
            pl.BlockSpec((2, D), lambda i: (0, 0)),
        ],
        out_shape=[
            jax.ShapeDtypeStruct((N, D), jnp.float32),
            jax.ShapeDtypeStruct((2, D), jnp.float32),
        ],
        scratch_shapes=[pltpu.VMEM((2, D), jnp.float32)],
    )(h, parts, w1, b1, w2, b2)


def _bn_body(z_ref, stats_ref, g_ref, be_ref, o_ref):
    inv_n = jnp.float32(1.0 / N)
    mean = stats_ref[0:1] * inv_n
    var = stats_ref[1:2] * inv_n - mean * mean
    scale = g_ref[...] * lax.rsqrt(var + 1e-5)
    o_ref[...] = jnp.tanh((z_ref[...] - mean) * scale + be_ref[...])


def _bn(z, stats, g, be):
    return pl.pallas_call(
        _bn_body,
        grid=(NB,),
        in_specs=[
            pl.BlockSpec((BLK, D), lambda i: (i, 0)),
            pl.BlockSpec((2, D), lambda i: (0, 0)),
            pl.BlockSpec((1, D), lambda i: (0, 0)),
            pl.BlockSpec((1, D), lambda i: (0, 0)),
        ],
        out_specs=pl.BlockSpec((BLK, D), lambda i: (i, 0)),
        out_shape=jax.ShapeDtypeStruct((N, D), jnp.float32),
    )(z, stats, g, be)


def _build_tables(edge_index):
    src = edge_index[0]
    dst = edge_index[1]
    qid = (src >= B).astype(jnp.int32) * 2 + (dst >= B).astype(jnp.int32)
    lsrc = src - (qid // 2) * B
    ldst = dst - (qid % 2) * B
    perm = jnp.argsort(qid)
    ls = jnp.concatenate([lsrc[perm], jnp.zeros((CAPQ,), jnp.int32)])
    ld = jnp.concatenate([ldst[perm], jnp.full((CAPQ,), DUMMY, jnp.int32)])
    nq = jnp.sum(qid[:, None] == jnp.arange(NQ, dtype=jnp.int32)[None, :],
                 axis=0, dtype=jnp.int32)
    cum = jnp.concatenate([jnp.zeros((1,), jnp.int32), jnp.cumsum(nq)[:-1]])
    pos = jnp.arange(CAPQ, dtype=jnp.int32)
    srcs, dsts = [], []
    for q in range(NQ):
        seg_s = lax.dynamic_slice(ls, (cum[q],), (CAPQ,))
        seg_d = lax.dynamic_slice(ld, (cum[q],), (CAPQ,))
        mask = pos < nq[q]
        srcs.append(jnp.where(mask, seg_s, 0))
        dsts.append(jnp.where(mask, seg_d, DUMMY))
    tbl_src = jnp.stack(srcs)
    tbl_dst = jnp.stack(dsts)
    return (tbl_src.reshape(NQ, NS, SLOTS, KQ),
            tbl_dst.reshape(NQ, NS, SLOTS, KQ))


def kernel(x, edge_index, W0, b0, W1, B1, W2, B2, G, Be):
    tbl_src, tbl_dst = _build_tables(edge_index)
    zeros_a = jnp.zeros((ASTRIPE, D), jnp.float32)

    h = _linear(x, W0, b0.reshape(1, D))
    outs = [x]
    for l in range(L):
        parts = _agg(h, tbl_src, tbl_dst,
                     zeros_a).reshape(NC, 2, AGG_ROWS, D)
        z, stats = _mlp(h, parts, W1[l], B1[l].reshape(1, D),
                        W2[l], B2[l].reshape(1, D))
        h = _bn(z, stats, G[l].reshape(1, D), Be[l].reshape(1, D))
        outs.append(h)
    return tuple(outs)


# speedup vs baseline: 3.2113x; 1.0522x over previous
import functools

import jax
import jax.numpy as jnp
from jax import lax
from jax.experimental import pallas as pl
from jax.experimental.pallas import tpu as pltpu
from jax.experimental.pallas import tpu_sc as plsc

N = 10000
E = 320000
D = 128
L = 3

NC = 2
NS = 16
NW = NC * NS

B = 5000
KQ = 128
SLOTS = 48
NQ = 4
CAPQ = NS * SLOTS * KQ

HS_ROWS = 5120
HSTRIPE = HS_ROWS // NS
AGG_ROWS = 5248
ASTRIPE = AGG_ROWS // NS
DUMMY = 5100

BLK = 1000
NB = N // BLK


def _agg_body(h_hbm, src_hbm, dst_hbm, zeros_hbm, out_hbm,
              src_t, dst_t, rows0, rows1, h_sh, agg_sh,
              hsem, isem, zsem, gsem0, gsem1, ssem0, ssem1):
    c = lax.axis_index("c")
    s = lax.axis_index("s")
    hoff = c * B + s * 312

    def _h_copy(rows):
        return pltpu.make_async_copy(h_hbm.at[pl.ds(hoff, rows)],
                                     h_sh.at[pl.ds(s * 312, rows)], hsem)

    @pl.when(s < NS - 1)
    def _():
        _h_copy(312).start()

    @pl.when(s == NS - 1)
    def _():
        _h_copy(320).start()

    bufs = (rows0, rows1)
    gsems = (gsem0, gsem1)
    ssems = (ssem0, ssem1)

    def start_gather(b, j):
        pltpu.async_copy(h_sh.at[src_t.at[j]], bufs[b], gsems[b])

    def wait_gather(b, j):
        pltpu.make_async_copy(h_sh.at[src_t.at[j]], bufs[b], gsems[b]).wait()

    def start_scatter(b, j):
        pltpu.async_copy(bufs[b], agg_sh.at[dst_t.at[j]], ssems[b], add=True)

    def wait_scatter(b, j):
        pltpu.make_async_copy(bufs[b], agg_sh.at[dst_t.at[j]],
                              ssems[b]).wait()

    def run_pass(p):
        q = c * 2 + p
        az = agg_sh.at[pl.ds(s * ASTRIPE, ASTRIPE)]
        zc = pltpu.make_async_copy(zeros_hbm, az, zsem)
        sc_ = pltpu.make_async_copy(src_hbm.at[q].at[s], src_t, isem)
        dc_ = pltpu.make_async_copy(dst_hbm.at[q].at[s], dst_t, isem)
        zc.start()
        sc_.start()
        dc_.start()
        zc.wait()
        sc_.wait()
        dc_.wait()
        plsc.subcore_barrier()

        start_gather(0, 0)

        def body(j, carry):
            def step(cur, oth):
                wait_gather(cur, j)
                start_scatter(cur, j)

                @pl.when(j >= 1)
                def _():
                    wait_scatter(oth, j - 1)

                @pl.when(j + 1 < SLOTS)
                def _():
                    start_gather(oth, j + 1)

            @pl.when(j % 2 == 0)
            def _():
                step(0, 1)

            @pl.when(j % 2 == 1)
            def _():
                step(1, 0)

            return carry

        lax.fori_loop(0, SLOTS, body, 0)
        wait_scatter((SLOTS - 1) % 2, SLOTS - 1)
        plsc.subcore_barrier()
        off = q * AGG_ROWS + s * ASTRIPE
        pltpu.sync_copy(agg_sh.at[pl.ds(s * ASTRIPE, ASTRIPE)],
                        out_hbm.at[pl.ds(off, ASTRIPE)])

    @pl.when(s < NS - 1)
    def _():
        _h_copy(312).wait()

    @pl.when(s == NS - 1)
    def _():
        _h_copy(320).wait()

    run_pass(0)
    run_pass(1)


@functools.cache
def _agg_kernel():
    return pl.kernel(
        _agg_body,
        out_type=jax.ShapeDtypeStruct((NC * 2 * AGG_ROWS, D), jnp.float32),
        mesh=plsc.VectorSubcoreMesh(core_axis_name="c", subcore_axis_name="s",
                                    num_cores=NC, num_subcores=NS),
        scratch_types=[
            pltpu.VMEM((SLOTS, KQ), jnp.int32),
            pltpu.VMEM((SLOTS, KQ), jnp.int32),
            pltpu.VMEM((KQ, D), jnp.float32),
            pltpu.VMEM((KQ, D), jnp.float32),
            pltpu.VMEM_SHARED((HS_ROWS, D), jnp.float32),
            pltpu.VMEM_SHARED((AGG_ROWS, D), jnp.float32),
            pltpu.SemaphoreType.DMA,
            pltpu.SemaphoreType.DMA,
            pltpu.SemaphoreType.DMA,
            pltpu.SemaphoreType.DMA,
            pltpu.SemaphoreType.DMA,
            pltpu.SemaphoreType.DMA,
            pltpu.SemaphoreType.DMA,
        ],
    )


def _agg(h, tbl_src, tbl_dst, zeros_a):
    return _agg_kernel()(h, tbl_src, tbl_dst, zeros_a)


def _lin_body(x_ref, w_ref, b_ref, o_ref):
    o_ref[...] = (
        jnp.dot(x_ref[...], w_ref[...], preferred_element_type=jnp.float32)
        + b_ref[...]
    )


def _linear(x, w, b):
    return pl.pallas_call(
        _lin_body,
        grid=(NB,),
        in_specs=[
            pl.BlockSpec((BLK, D), lambda i: (i, 0)),
            pl.BlockSpec((D, D), lambda i: (0, 0)),
            pl.BlockSpec((1, D), lambda i: (0, 0)),
        ],
        out_specs=pl.BlockSpec((BLK, D), lambda i: (i, 0)),
        out_shape=jax.ShapeDtypeStruct((N, D), jnp.float32),
    )(x, w, b)


def _mlp_body(h_ref, agg_ref, w1_ref, b1_ref, w2_ref, b2_ref,
              z_ref, stats_ref, acc_ref):
    i = pl.program_id(0)
    z = h_ref[...] + agg_ref[0, 0] + agg_ref[1, 0]
    z = jnp.maximum(
        jnp.dot(z, w1_ref[...], preferred_element_type=jnp.float32)
        + b1_ref[...], 0.0)
    z = jnp.maximum(
        jnp.dot(z, w2_ref[...], preferred_element_type=jnp.float32)
        + b2_ref[...], 0.0)
    z_ref[...] = z

    @pl.when(i == 0)
    def _():
        acc_ref[...] = jnp.zeros_like(acc_ref)

    acc_ref[0:1] += jnp.sum(z, axis=0, keepdims=True)
    acc_ref[1:2] += jnp.sum(z * z, axis=0, keepdims=True)
    stats_ref[...] = acc_ref[...]


def _mlp(h, parts, w1, b1, w2, b2):
    nblk = B // BLK
    return pl.pallas_call(
        _mlp_body,
        grid=(NB,),
        in_specs=[
            pl.BlockSpec((BLK, D), lambda i: (i, 0)),
            pl.BlockSpec((NC, 1, BLK, D),
                         lambda i: (0, i // nblk, i % nblk, 0)),
            pl.BlockSpec((D, D), lambda i: (0, 0)),
            pl.BlockSpec((1, D), lambda i: (0, 0)),
            pl.BlockSpec((D, D), lambda i: (0, 0)),
            pl.BlockSpec((1, D), lambda i: (0, 0)),
        ],
        out_specs=[
            pl.BlockSpec((BLK, D), lambda i: (i, 0)),
            pl.BlockSpec((2, D), lambda i: (0, 0)),
        ],
        out_shape=[
            jax.ShapeDtypeStruct((N, D), jnp.float32),
            jax.ShapeDtypeStruct((2, D), jnp.float32),
        ],
        scratch_shapes=[pltpu.VMEM((2, D), jnp.float32)],
    )(h, parts, w1, b1, w2, b2)


def _bn_body(z_ref, stats_ref, g_ref, be_ref, o_ref):
    inv_n = jnp.float32(1.0 / N)
    mean = stats_ref[0:1] * inv_n
    var = stats_ref[1:2] * inv_n - mean * mean
    scale = g_ref[...] * lax.rsqrt(var + 1e-5)
    o_ref[...] = jnp.tanh((z_ref[...] - mean) * scale + be_ref[...])


def _bn(z, stats, g, be):
    return pl.pallas_call(
        _bn_body,
        grid=(NB,),
        in_specs=[
            pl.BlockSpec((BLK, D), lambda i: (i, 0)),
            pl.BlockSpec((2, D), lambda i: (0, 0)),
            pl.BlockSpec((1, D), lambda i: (0, 0)),
            pl.BlockSpec((1, D), lambda i: (0, 0)),
        ],
        out_specs=pl.BlockSpec((BLK, D), lambda i: (i, 0)),
        out_shape=jax.ShapeDtypeStruct((N, D), jnp.float32),
    )(z, stats, g, be)


def _build_tables(edge_index):
    src = edge_index[0]
    dst = edge_index[1]
    qid = (src >= B).astype(jnp.int32) * 2 + (dst >= B).astype(jnp.int32)
    lsrc = src - (qid // 2) * B
    ldst = dst - (qid % 2) * B
    packed = jnp.sort((qid << 28) | (lsrc << 14) | ldst)
    ls = jnp.concatenate([(packed >> 14) & 0x3FFF, jnp.zeros((CAPQ,), jnp.int32)])
    ld = jnp.concatenate([packed & 0x3FFF, jnp.full((CAPQ,), DUMMY, jnp.int32)])
    nq = jnp.sum(qid[:, None] == jnp.arange(NQ, dtype=jnp.int32)[None, :],
                 axis=0, dtype=jnp.int32)
    cum = jnp.concatenate([jnp.zeros((1,), jnp.int32), jnp.cumsum(nq)[:-1]])
    pos = jnp.arange(CAPQ, dtype=jnp.int32)
    srcs, dsts = [], []
    for q in range(NQ):
        seg_s = lax.dynamic_slice(ls, (cum[q],), (CAPQ,))
        seg_d = lax.dynamic_slice(ld, (cum[q],), (CAPQ,))
        mask = pos < nq[q]
        srcs.append(jnp.where(mask, seg_s, 0))
        dsts.append(jnp.where(mask, seg_d, DUMMY))
    tbl_src = jnp.stack(srcs)
    tbl_dst = jnp.stack(dsts)
    return (tbl_src.reshape(NQ, NS, SLOTS, KQ),
            tbl_dst.reshape(NQ, NS, SLOTS, KQ))


def kernel(x, edge_index, W0, b0, W1, B1, W2, B2, G, Be):
    tbl_src, tbl_dst = _build_tables(edge_index)
    zeros_a = jnp.zeros((ASTRIPE, D), jnp.float32)

    h = _linear(x, W0, b0.reshape(1, D))
    outs = [x]
    for l in range(L):
        parts = _agg(h, tbl_src, tbl_dst,
                     zeros_a).reshape(NC, 2, AGG_ROWS, D)
        z, stats = _mlp(h, parts, W1[l], B1[l].reshape(1, D),
                        W2[l], B2[l].reshape(1, D))
        h = _bn(z, stats, G[l].reshape(1, D), Be[l].reshape(1, D))
        outs.append(h)
    return tuple(outs)
